# baseline (speedup 1.0000x reference)
import functools

import jax
import jax.numpy as jnp
from jax.experimental import pallas as pl
from jax.experimental.pallas import tpu as pltpu

_BM = 4608
_BK = 2048


def _vq_loss_kernel(zt_ref, c_ref, out_ref, csq_ref, *, n_tokens, k_codes):
    @pl.when(pl.program_id(0) == 0)
    def _csq():
        c = c_ref[:]
        csq_ref[:] = jnp.sum(c * c, axis=1, keepdims=True)

    ztb = zt_ref[:]
    z2 = -2.0 * ztb

    def body(i, minv):
        cb = c_ref[pl.ds(i * _BK, _BK), :]
        dots = jnp.dot(cb, z2, preferred_element_type=jnp.float32)
        part = csq_ref[pl.ds(i * _BK, _BK), :] + dots
        return jnp.minimum(minv, jnp.min(part, axis=0, keepdims=True))

    minv = jax.lax.fori_loop(
        0, k_codes // _BK, body,
        jnp.full((1, ztb.shape[1]), jnp.inf, dtype=jnp.float32),
        unroll=4)
    zsq = jnp.sum(ztb * ztb, axis=0, keepdims=True)
    s = jnp.sum(minv + zsq)

    @pl.when(pl.program_id(0) == 0)
    def _init():
        out_ref[0, 0] = 0.0

    out_ref[0, 0] += s / n_tokens


def kernel(z, codebook):
    n, d = z.shape
    k = codebook.shape[0]
    zt = z.T
    out = pl.pallas_call(
        functools.partial(_vq_loss_kernel, n_tokens=n, k_codes=k),
        grid=(n // _BM,),
        in_specs=[
            pl.BlockSpec((d, _BM), lambda m: (0, m)),
            pl.BlockSpec((k, d), lambda m: (0, 0)),
        ],
        out_specs=pl.BlockSpec(memory_space=pltpu.SMEM),
        out_shape=jax.ShapeDtypeStruct((1, 1), jnp.float32),
        scratch_shapes=[pltpu.VMEM((k, 1), jnp.float32)],
    )(zt, codebook)
    return out[0, 0]
